# baseline (speedup 1.0000x reference)
import functools

import jax
import jax.numpy as jnp
from jax import lax
from jax.experimental import pallas as pl
from jax.experimental.pallas import tpu as pltpu
from jax.experimental.pallas import tpu_sc as plsc

N = 10000
E = 320000
D_IN = 128
D_H = 64
G = 256
T = 5
EPS = 1e-5

NC = 2
NS = 16
NW = NC * NS
C = 128
K = 2
EPT = 10240
NCHUNK = EPT // C
NSUP = NCHUNK // K
EP = EPT * NW
NP = 10240
RPT = NP // NS

@functools.lru_cache(maxsize=None)
def _sc_edge_scatter(width, do_gather):
  fill = 0.0 if do_gather else 1.0

  def body(src_hbm, dst_hbm, table_hbm, out_hbm, src_v, dst_v, rows_v,
           fill_v, acc_sh, gsem0, gsem1, ssem0, ssem1):
    c = lax.axis_index("c")
    s = lax.axis_index("s")
    w = c * NS + s
    gsem = (gsem0, gsem1)
    ssem = (ssem0, ssem1)

    def fill_row(i, _):
      for j in range(width // 16):
        fill_v[i, pl.ds(j * 16, 16)] = jnp.full((16,), fill, jnp.float32)
      return 0
    lax.fori_loop(0, C, fill_row, 0)

    if do_gather:
      zsrc = fill_v
    else:
      zsrc = rows_v.at[0, pl.ds(0, C)]
      def zero_row(i, _):
        rows_v[0, i, pl.ds(0, 16)] = jnp.zeros((16,), jnp.float32)
        return 0
      lax.fori_loop(0, C, zero_row, 0)
    for k in range(RPT // C):
      pltpu.sync_copy(zsrc, acc_sh.at[pl.ds(s * RPT + k * C, C)])
    plsc.subcore_barrier()

    pltpu.sync_copy(dst_hbm.at[w], dst_v)

    if do_gather:
      pltpu.sync_copy(
          src_hbm.at[pl.ds(pl.multiple_of(w * EPT, EPT), EPT)], src_v)

      def fire_gathers(t, a):
        for k in range(K):
          off = pl.multiple_of((t * K + k) * C, C)
          pltpu.async_copy(table_hbm.at[src_v.at[pl.ds(off, C)]],
                           rows_v.at[a, pl.ds(k * C, C)], gsem[a])

      def drain_gathers(a):
        pltpu.make_async_copy(table_hbm.at[pl.ds(0, K * C)],
                              rows_v.at[a], gsem[a]).wait()

      def fire_scatters(t, a):
        for k in range(K):
          pltpu.async_copy(rows_v.at[a, pl.ds(k * C, C)],
                           acc_sh.at[dst_v.at[t * K + k]], ssem[a], add=True)

      def drain_scatters(a):
        pltpu.make_async_copy(rows_v.at[a],
                              acc_sh.at[pl.ds(0, K * C)], ssem[a]).wait()

      fire_gathers(0, 0)

      def sup_pair(i, _):
        for a in (0, 1):
          t = 2 * i + a
          b = 1 - a

          @pl.when(t + 1 < NSUP)
          def _():
            @pl.when(t >= 1)
            def _():
              drain_scatters(b)
            fire_gathers(t + 1, b)

          drain_gathers(a)
          fire_scatters(t, a)
        return 0

      lax.fori_loop(0, NSUP // 2, sup_pair, 0)
      drain_scatters(0)
      drain_scatters(1)
    else:
      def chunk(j, _):
        pltpu.sync_copy(fill_v, acc_sh.at[dst_v.at[j]], add=True)
        return 0
      lax.fori_loop(0, NCHUNK, chunk, 0)

    plsc.subcore_barrier()
    pltpu.sync_copy(acc_sh.at[pl.ds(s * RPT, RPT)],
                    out_hbm.at[c, pl.ds(s * RPT, RPT)])

  mesh = plsc.VectorSubcoreMesh(
      core_axis_name="c", subcore_axis_name="s",
      num_cores=NC, num_subcores=NS)
  kern = functools.partial(
      pl.kernel,
      out_type=jax.ShapeDtypeStruct((NC, NP, width), jnp.float32),
      mesh=mesh,
      compiler_params=pltpu.CompilerParams(use_tc_tiling_on_sc=False),
      scratch_types=[
          pltpu.VMEM((EPT,), jnp.int32),
          pltpu.VMEM((NCHUNK, C), jnp.int32),
          pltpu.VMEM((2, K * C, width), jnp.float32),
          pltpu.VMEM((C, width), jnp.float32),
          pltpu.VMEM_SHARED((NP, width), jnp.float32),
          pltpu.SemaphoreType.DMA,
          pltpu.SemaphoreType.DMA,
          pltpu.SemaphoreType.DMA,
          pltpu.SemaphoreType.DMA,
      ],
  )(body)
  return kern


def _sc_count(src, dst, table):
  return _sc_edge_scatter(16, False)(src, dst, table)


def _sc_gather_scatter(src, dst, table):
  return _sc_edge_scatter(D_H, True)(src, dst, table)


_ROWS_BLK = 256
_NBLK = NP // _ROWS_BLK


def _k0_body(x_ref, w1_ref, degp_ref, h1_ref, dinv_ref):
  deg = degp_ref[0] + degp_ref[1] + 1.0
  dinv = lax.rsqrt(jnp.maximum(deg, 1.0))
  dinv_ref[...] = dinv
  h = jnp.dot(x_ref[...], w1_ref[...], preferred_element_type=jnp.float32)
  h1_ref[...] = h * dinv[:, :1]


def _kmid_body(p_ref, h_ref, dinv_ref, b_ref, w_ref, out_ref):
  dinv = dinv_ref[:, :1]
  t = dinv * (p_ref[0] + p_ref[1] + h_ref[...]) + b_ref[...]
  t = jnp.maximum(t, 0.0)
  out_ref[...] = jnp.dot(
      t, w_ref[...], preferred_element_type=jnp.float32) * dinv


def _kfinal_body(p_ref, h_ref, dinv_ref, b_ref, out_ref):
  dinv = dinv_ref[:, :1]
  t = dinv * (p_ref[0] + p_ref[1] + h_ref[...]) + b_ref[...]
  out_ref[...] = jnp.maximum(t, 0.0)


def _kpool_body(starts_ref, t3_ref, emb_ref):
  g = pl.program_id(0)
  s0 = starts_ref[g]
  e0 = starts_ref[g + 1]
  base = (s0 // 8) * 8
  nblk = (e0 - base + 7) // 8

  def body(i, carry):
    sacc, macc = carry
    r0 = base + i * 8
    rows = t3_ref[pl.ds(pl.multiple_of(r0, 8), 8), :]
    rid = r0 + lax.broadcasted_iota(jnp.int32, (8, 1), 0)
    m = (rid >= s0) & (rid < e0)
    sacc = sacc + jnp.where(m, rows, 0.0)
    macc = jnp.maximum(macc, jnp.where(m, rows, -jnp.inf))
    return sacc, macc

  init = (jnp.zeros((8, D_H), jnp.float32),
          jnp.full((8, D_H), -jnp.inf, jnp.float32))
  sacc, macc = lax.fori_loop(0, nblk, body, init)
  cnt = jnp.maximum((e0 - s0).astype(jnp.float32), 1.0)
  emb_ref[0, :, :D_H] = jnp.sum(sacc, axis=0, keepdims=True) / cnt
  emb_ref[0, :, D_H:] = jnp.max(macc, axis=0, keepdims=True)


def _kmlp_body(emb_ref, wc1_ref, bc1_ref, s1_ref, t1_ref,
               wc2_ref, bc2_ref, s2_ref, t2_ref, wc3_ref, bc3_ref, out_ref):
  z = jnp.dot(emb_ref[...], wc1_ref[...], preferred_element_type=jnp.float32)
  z = jnp.maximum(z * s1_ref[...] + t1_ref[...], 0.0)
  z = jnp.dot(z, wc2_ref[...], preferred_element_type=jnp.float32)
  z = jnp.maximum(z * s2_ref[...] + t2_ref[...], 0.0)
  out_ref[...] = jnp.dot(
      z, wc3_ref[...], preferred_element_type=jnp.float32) + bc3_ref[...]


def kernel(x, edge_index, batch, W1, b1, W2, b2, W3, b3, Wc1, bc1, g1, be1,
           m1, v1, Wc2, bc2, g2, be2, m2, v2, Wc3, bc3):
  f32 = jnp.float32
  pad_e = EP - E
  pad_dst = N + jnp.arange(pad_e, dtype=jnp.int32) % (NP - N)
  src = jnp.concatenate([edge_index[0], jnp.zeros((pad_e,), jnp.int32)])
  dst = jnp.concatenate(
      [edge_index[1], pad_dst]).reshape(NW, NCHUNK, C)
  xp = jnp.pad(x, ((0, NP - N), (0, 0)))
  starts = jnp.searchsorted(batch, jnp.arange(G + 1, dtype=jnp.int32)
                            ).astype(jnp.int32)

  degp = _sc_count(src, dst, jnp.zeros((NP, 16), f32))

  blk = _ROWS_BLK
  h1, dinv = pl.pallas_call(
      _k0_body,
      grid=(_NBLK,),
      in_specs=[
          pl.BlockSpec((blk, D_IN), lambda i: (i, 0)),
          pl.BlockSpec((D_IN, D_H), lambda i: (0, 0)),
          pl.BlockSpec((NC, blk, 16), lambda i: (0, i, 0)),
      ],
      out_specs=[
          pl.BlockSpec((blk, D_H), lambda i: (i, 0)),
          pl.BlockSpec((blk, 16), lambda i: (i, 0)),
      ],
      out_shape=[
          jax.ShapeDtypeStruct((NP, D_H), f32),
          jax.ShapeDtypeStruct((NP, 16), f32),
      ],
  )(xp, W1, degp)

  def mid(h, b, w):
    p = _sc_gather_scatter(src, dst, h)
    return pl.pallas_call(
        _kmid_body,
        grid=(_NBLK,),
        in_specs=[
            pl.BlockSpec((NC, blk, D_H), lambda i: (0, i, 0)),
            pl.BlockSpec((blk, D_H), lambda i: (i, 0)),
            pl.BlockSpec((blk, 16), lambda i: (i, 0)),
            pl.BlockSpec((1, D_H), lambda i: (0, 0)),
            pl.BlockSpec((D_H, D_H), lambda i: (0, 0)),
        ],
        out_specs=pl.BlockSpec((blk, D_H), lambda i: (i, 0)),
        out_shape=jax.ShapeDtypeStruct((NP, D_H), f32),
    )(p, h, dinv, b.reshape(1, D_H), w)

  h2 = mid(h1, b1, W2)
  h3 = mid(h2, b2, W3)

  p3 = _sc_gather_scatter(src, dst, h3)
  t3 = pl.pallas_call(
      _kfinal_body,
      grid=(_NBLK,),
      in_specs=[
          pl.BlockSpec((NC, blk, D_H), lambda i: (0, i, 0)),
          pl.BlockSpec((blk, D_H), lambda i: (i, 0)),
          pl.BlockSpec((blk, 16), lambda i: (i, 0)),
          pl.BlockSpec((1, D_H), lambda i: (0, 0)),
      ],
      out_specs=pl.BlockSpec((blk, D_H), lambda i: (i, 0)),
      out_shape=jax.ShapeDtypeStruct((NP, D_H), f32),
  )(p3, h3, dinv, b3.reshape(1, D_H))

  emb = pl.pallas_call(
      _kpool_body,
      grid=(G,),
      in_specs=[
          pl.BlockSpec(memory_space=pltpu.SMEM),
          pl.BlockSpec((NP, D_H), lambda g: (0, 0)),
      ],
      out_specs=pl.BlockSpec((1, 1, 2 * D_H), lambda g: (g, 0, 0)),
      out_shape=jax.ShapeDtypeStruct((G, 1, 2 * D_H), f32),
  )(starts, t3)
  emb = emb.reshape(G, 2 * D_H)

  s1 = (g1 * lax.rsqrt(v1 + EPS)).reshape(1, D_H)
  t1 = (be1 - m1 * g1 * lax.rsqrt(v1 + EPS) + bc1 * g1 * lax.rsqrt(v1 + EPS)
        ).reshape(1, D_H)
  s2 = (g2 * lax.rsqrt(v2 + EPS)).reshape(1, D_H // 2)
  t2 = (be2 - m2 * g2 * lax.rsqrt(v2 + EPS) + bc2 * g2 * lax.rsqrt(v2 + EPS)
        ).reshape(1, D_H // 2)

  full = lambda shape: pl.BlockSpec(shape, lambda: (0,) * len(shape))
  out = pl.pallas_call(
      _kmlp_body,
      in_specs=[
          full((G, 2 * D_H)), full((2 * D_H, D_H)), full((1, D_H)),
          full((1, D_H)), full((1, D_H)),
          full((D_H, D_H // 2)), full((1, D_H // 2)), full((1, D_H // 2)),
          full((1, D_H // 2)), full((D_H // 2, T)), full((1, T)),
      ],
      out_specs=full((G, T)),
      out_shape=jax.ShapeDtypeStruct((G, T), f32),
  )(emb, Wc1, bc1.reshape(1, D_H), s1, t1,
    Wc2, bc2.reshape(1, D_H // 2), s2, t2, Wc3, bc3.reshape(1, T))
  return out


# speedup vs baseline: 1.0750x; 1.0750x over previous
import functools

import jax
import jax.numpy as jnp
from jax import lax
from jax.experimental import pallas as pl
from jax.experimental.pallas import tpu as pltpu
from jax.experimental.pallas import tpu_sc as plsc

N = 10000
E = 320000
D_IN = 128
D_H = 64
G = 256
T = 5
EPS = 1e-5

NC = 2
NS = 16
NW = NC * NS
C = 128
K = 2
TOTCH = 2560
EP = TOTCH * C
SLOW_CORE = 0
NCH_SLOW = 40
NCH_FAST = 120
NP = 10240
RPT = NP // NS

@functools.lru_cache(maxsize=None)
def _sc_edge_scatter(width, do_gather):
  fill = 0.0 if do_gather else 1.0

  def body(src_hbm, dst_hbm, table_hbm, out_hbm, src_v, dst_v, rows_v,
           fill_v, acc_sh, gsem0, gsem1, ssem0, ssem1):
    c = lax.axis_index("c")
    s = lax.axis_index("s")
    gsem = (gsem0, gsem1)
    ssem = (ssem0, ssem1)

    def fill_row(i, _):
      for j in range(width // 16):
        fill_v[i, pl.ds(j * 16, 16)] = jnp.full((16,), fill, jnp.float32)
      return 0
    lax.fori_loop(0, C, fill_row, 0)

    if do_gather:
      zsrc = fill_v
    else:
      zsrc = rows_v.at[0, pl.ds(0, C)]
      def zero_row(i, _):
        rows_v[0, i, pl.ds(0, 16)] = jnp.zeros((16,), jnp.float32)
        return 0
      lax.fori_loop(0, C, zero_row, 0)
    for k in range(RPT // C):
      pltpu.sync_copy(zsrc, acc_sh.at[pl.ds(s * RPT + k * C, C)])
    plsc.subcore_barrier()

    def run_edges(nch, chunk_base):
      pltpu.sync_copy(dst_hbm.at[pl.ds(chunk_base, nch)],
                      dst_v.at[pl.ds(0, nch)])
      if not do_gather:
        def chunk(j, _):
          pltpu.sync_copy(fill_v, acc_sh.at[dst_v.at[j]], add=True)
          return 0
        lax.fori_loop(0, nch, chunk, 0)
        return

      nsup = nch // K
      pltpu.sync_copy(
          src_hbm.at[pl.ds(pl.multiple_of(chunk_base * C, C), nch * C)],
          src_v.at[pl.ds(0, nch * C)])

      def fire_gathers(t, a):
        for k in range(K):
          off = pl.multiple_of((t * K + k) * C, C)
          pltpu.async_copy(table_hbm.at[src_v.at[pl.ds(off, C)]],
                           rows_v.at[a, pl.ds(k * C, C)], gsem[a])

      def drain_gathers(a):
        pltpu.make_async_copy(table_hbm.at[pl.ds(0, K * C)],
                              rows_v.at[a], gsem[a]).wait()

      def fire_scatters(t, a):
        for k in range(K):
          pltpu.async_copy(rows_v.at[a, pl.ds(k * C, C)],
                           acc_sh.at[dst_v.at[t * K + k]], ssem[a], add=True)

      def drain_scatters(a):
        pltpu.make_async_copy(rows_v.at[a],
                              acc_sh.at[pl.ds(0, K * C)], ssem[a]).wait()

      fire_gathers(0, 0)

      def sup_pair(i, _):
        for a in (0, 1):
          t = 2 * i + a
          b = 1 - a

          @pl.when(t + 1 < nsup)
          def _():
            @pl.when(t >= 1)
            def _():
              drain_scatters(b)
            fire_gathers(t + 1, b)

          drain_gathers(a)
          fire_scatters(t, a)
        return 0

      lax.fori_loop(0, nsup // 2, sup_pair, 0)
      drain_scatters(0)
      drain_scatters(1)

    if SLOW_CORE == 0:
      sizes = ((NCH_SLOW, 0), (NCH_FAST, NS * NCH_SLOW))
    else:
      sizes = ((NCH_FAST, 0), (NCH_SLOW, NS * NCH_FAST))
    for core_val, (nch, core_base) in enumerate(sizes):
      @pl.when(c == core_val)
      def _(nch=nch, core_base=core_base):
        run_edges(nch, core_base + s * nch)

    plsc.subcore_barrier()
    pltpu.sync_copy(acc_sh.at[pl.ds(s * RPT, RPT)],
                    out_hbm.at[c, pl.ds(s * RPT, RPT)])

  mesh = plsc.VectorSubcoreMesh(
      core_axis_name="c", subcore_axis_name="s",
      num_cores=NC, num_subcores=NS)
  kern = functools.partial(
      pl.kernel,
      out_type=jax.ShapeDtypeStruct((NC, NP, width), jnp.float32),
      mesh=mesh,
      compiler_params=pltpu.CompilerParams(use_tc_tiling_on_sc=False),
      scratch_types=[
          pltpu.VMEM((NCH_FAST * C,), jnp.int32),
          pltpu.VMEM((NCH_FAST, C), jnp.int32),
          pltpu.VMEM((2, K * C, width), jnp.float32),
          pltpu.VMEM((C, width), jnp.float32),
          pltpu.VMEM_SHARED((NP, width), jnp.float32),
          pltpu.SemaphoreType.DMA,
          pltpu.SemaphoreType.DMA,
          pltpu.SemaphoreType.DMA,
          pltpu.SemaphoreType.DMA,
      ],
  )(body)
  return kern


def _sc_count(src, dst, table):
  return _sc_edge_scatter(16, False)(src, dst, table)


def _sc_gather_scatter(src, dst, table):
  return _sc_edge_scatter(D_H, True)(src, dst, table)


_ROWS_BLK = 512
_NBLK = NP // _ROWS_BLK


def _k0_body(x_ref, w1_ref, degp_ref, h1_ref, dinv_ref):
  deg = degp_ref[0] + degp_ref[1] + 1.0
  dinv = lax.rsqrt(jnp.maximum(deg, 1.0))
  dinv_ref[...] = dinv
  h = jnp.dot(x_ref[...], w1_ref[...], preferred_element_type=jnp.float32)
  h1_ref[...] = h * dinv[:, :1]


def _kmid_body(p_ref, h_ref, dinv_ref, b_ref, w_ref, out_ref):
  dinv = dinv_ref[:, :1]
  t = dinv * (p_ref[0] + p_ref[1] + h_ref[...]) + b_ref[...]
  t = jnp.maximum(t, 0.0)
  out_ref[...] = jnp.dot(
      t, w_ref[...], preferred_element_type=jnp.float32) * dinv


def _kfinal_body(p_ref, h_ref, dinv_ref, b_ref, out_ref):
  dinv = dinv_ref[:, :1]
  t = dinv * (p_ref[0] + p_ref[1] + h_ref[...]) + b_ref[...]
  out_ref[...] = jnp.maximum(t, 0.0)


_G_PER = 8


def _kpool_body(starts_ref, t3_ref, emb_ref):
  for gg in range(_G_PER):
    g = pl.program_id(0) * _G_PER + gg
    s0 = starts_ref[g]
    e0 = starts_ref[g + 1]
    base = (s0 // 8) * 8
    nblk = (e0 - base + 7) // 8

    def body(i, carry):
      sacc, macc = carry
      r0 = base + i * 8
      rows = t3_ref[pl.ds(pl.multiple_of(r0, 8), 8), :]
      rid = r0 + lax.broadcasted_iota(jnp.int32, (8, 1), 0)
      m = (rid >= s0) & (rid < e0)
      sacc = sacc + jnp.where(m, rows, 0.0)
      macc = jnp.maximum(macc, jnp.where(m, rows, -jnp.inf))
      return sacc, macc

    init = (jnp.zeros((8, D_H), jnp.float32),
            jnp.full((8, D_H), -jnp.inf, jnp.float32))
    sacc, macc = lax.fori_loop(0, nblk, body, init)
    cnt = jnp.maximum((e0 - s0).astype(jnp.float32), 1.0)
    emb_ref[pl.ds(gg, 1), :D_H] = jnp.sum(sacc, axis=0, keepdims=True) / cnt
    emb_ref[pl.ds(gg, 1), D_H:] = jnp.max(macc, axis=0, keepdims=True)


def _kmlp_body(emb_ref, wc1_ref, bc1_ref, s1_ref, t1_ref,
               wc2_ref, bc2_ref, s2_ref, t2_ref, wc3_ref, bc3_ref, out_ref):
  z = jnp.dot(emb_ref[...], wc1_ref[...], preferred_element_type=jnp.float32)
  z = jnp.maximum(z * s1_ref[...] + t1_ref[...], 0.0)
  z = jnp.dot(z, wc2_ref[...], preferred_element_type=jnp.float32)
  z = jnp.maximum(z * s2_ref[...] + t2_ref[...], 0.0)
  out_ref[...] = jnp.dot(
      z, wc3_ref[...], preferred_element_type=jnp.float32) + bc3_ref[...]


def kernel(x, edge_index, batch, W1, b1, W2, b2, W3, b3, Wc1, bc1, g1, be1,
           m1, v1, Wc2, bc2, g2, be2, m2, v2, Wc3, bc3):
  f32 = jnp.float32
  pad_e = EP - E
  pad_dst = N + jnp.arange(pad_e, dtype=jnp.int32) % (NP - N)
  src = jnp.concatenate([edge_index[0], jnp.zeros((pad_e,), jnp.int32)])
  dst = jnp.concatenate(
      [edge_index[1], pad_dst]).reshape(TOTCH, C)
  xp = jnp.pad(x, ((0, NP - N), (0, 0)))
  starts = jnp.searchsorted(batch, jnp.arange(G + 1, dtype=jnp.int32)
                            ).astype(jnp.int32)

  degp = _sc_count(src, dst, jnp.zeros((NP, 16), f32))

  blk = _ROWS_BLK
  h1, dinv = pl.pallas_call(
      _k0_body,
      grid=(_NBLK,),
      in_specs=[
          pl.BlockSpec((blk, D_IN), lambda i: (i, 0)),
          pl.BlockSpec((D_IN, D_H), lambda i: (0, 0)),
          pl.BlockSpec((NC, blk, 16), lambda i: (0, i, 0)),
      ],
      out_specs=[
          pl.BlockSpec((blk, D_H), lambda i: (i, 0)),
          pl.BlockSpec((blk, 16), lambda i: (i, 0)),
      ],
      out_shape=[
          jax.ShapeDtypeStruct((NP, D_H), f32),
          jax.ShapeDtypeStruct((NP, 16), f32),
      ],
  )(xp, W1, degp)

  def mid(h, b, w):
    p = _sc_gather_scatter(src, dst, h)
    return pl.pallas_call(
        _kmid_body,
        grid=(_NBLK,),
        in_specs=[
            pl.BlockSpec((NC, blk, D_H), lambda i: (0, i, 0)),
            pl.BlockSpec((blk, D_H), lambda i: (i, 0)),
            pl.BlockSpec((blk, 16), lambda i: (i, 0)),
            pl.BlockSpec((1, D_H), lambda i: (0, 0)),
            pl.BlockSpec((D_H, D_H), lambda i: (0, 0)),
        ],
        out_specs=pl.BlockSpec((blk, D_H), lambda i: (i, 0)),
        out_shape=jax.ShapeDtypeStruct((NP, D_H), f32),
    )(p, h, dinv, b.reshape(1, D_H), w)

  h2 = mid(h1, b1, W2)
  h3 = mid(h2, b2, W3)

  p3 = _sc_gather_scatter(src, dst, h3)
  t3 = pl.pallas_call(
      _kfinal_body,
      grid=(_NBLK,),
      in_specs=[
          pl.BlockSpec((NC, blk, D_H), lambda i: (0, i, 0)),
          pl.BlockSpec((blk, D_H), lambda i: (i, 0)),
          pl.BlockSpec((blk, 16), lambda i: (i, 0)),
          pl.BlockSpec((1, D_H), lambda i: (0, 0)),
      ],
      out_specs=pl.BlockSpec((blk, D_H), lambda i: (i, 0)),
      out_shape=jax.ShapeDtypeStruct((NP, D_H), f32),
  )(p3, h3, dinv, b3.reshape(1, D_H))

  emb = pl.pallas_call(
      _kpool_body,
      grid=(G // _G_PER,),
      in_specs=[
          pl.BlockSpec(memory_space=pltpu.SMEM),
          pl.BlockSpec((NP, D_H), lambda g: (0, 0)),
      ],
      out_specs=pl.BlockSpec((_G_PER, 2 * D_H), lambda g: (g, 0)),
      out_shape=jax.ShapeDtypeStruct((G, 2 * D_H), f32),
  )(starts, t3)

  s1 = (g1 * lax.rsqrt(v1 + EPS)).reshape(1, D_H)
  t1 = (be1 - m1 * g1 * lax.rsqrt(v1 + EPS) + bc1 * g1 * lax.rsqrt(v1 + EPS)
        ).reshape(1, D_H)
  s2 = (g2 * lax.rsqrt(v2 + EPS)).reshape(1, D_H // 2)
  t2 = (be2 - m2 * g2 * lax.rsqrt(v2 + EPS) + bc2 * g2 * lax.rsqrt(v2 + EPS)
        ).reshape(1, D_H // 2)

  full = lambda shape: pl.BlockSpec(shape, lambda: (0,) * len(shape))
  out = pl.pallas_call(
      _kmlp_body,
      in_specs=[
          full((G, 2 * D_H)), full((2 * D_H, D_H)), full((1, D_H)),
          full((1, D_H)), full((1, D_H)),
          full((D_H, D_H // 2)), full((1, D_H // 2)), full((1, D_H // 2)),
          full((1, D_H // 2)), full((D_H // 2, T)), full((1, T)),
      ],
      out_specs=full((G, T)),
      out_shape=jax.ShapeDtypeStruct((G, T), f32),
  )(emb, Wc1, bc1.reshape(1, D_H), s1, t1,
    Wc2, bc2.reshape(1, D_H // 2), s2, t2, Wc3, bc3.reshape(1, T))
  return out


# speedup vs baseline: 1.1282x; 1.0494x over previous
import functools

import jax
import jax.numpy as jnp
from jax import lax
from jax.experimental import pallas as pl
from jax.experimental.pallas import tpu as pltpu
from jax.experimental.pallas import tpu_sc as plsc

N = 10000
E = 320000
D_IN = 128
D_H = 64
G = 256
T = 5
EPS = 1e-5

NC = 2
NS = 16
NW = NC * NS
C = 128
K = 2
TOTCH = 2560
EP = TOTCH * C
SLOW_CORE = 1
NCH_SLOW = 40
NCH_FAST = 120
NP = 10240
RPT = NP // NS

@functools.lru_cache(maxsize=None)
def _sc_edge_scatter(width, do_gather):
  fill = 0.0 if do_gather else 1.0

  def body(src_hbm, dst_hbm, table_hbm, out_hbm, src_v, dst_v, rows_v,
           fill_v, acc_sh, gsem0, gsem1, ssem0, ssem1):
    c = lax.axis_index("c")
    s = lax.axis_index("s")
    gsem = (gsem0, gsem1)
    ssem = (ssem0, ssem1)

    def fill_row(i, _):
      for j in range(width // 16):
        fill_v[i, pl.ds(j * 16, 16)] = jnp.full((16,), fill, jnp.float32)
      return 0
    lax.fori_loop(0, C, fill_row, 0)

    if do_gather:
      zsrc = fill_v
    else:
      zsrc = rows_v.at[0, pl.ds(0, C)]
      def zero_row(i, _):
        rows_v[0, i, pl.ds(0, 16)] = jnp.zeros((16,), jnp.float32)
        return 0
      lax.fori_loop(0, C, zero_row, 0)
    for k in range(RPT // C):
      pltpu.sync_copy(zsrc, acc_sh.at[pl.ds(s * RPT + k * C, C)])
    plsc.subcore_barrier()

    def run_edges(nch, chunk_base):
      pltpu.sync_copy(dst_hbm.at[pl.ds(chunk_base, nch)],
                      dst_v.at[pl.ds(0, nch)])
      if not do_gather:
        def chunk(j, _):
          pltpu.sync_copy(fill_v, acc_sh.at[dst_v.at[j]], add=True)
          return 0
        lax.fori_loop(0, nch, chunk, 0)
        return

      nsup = nch // K
      pltpu.sync_copy(
          src_hbm.at[pl.ds(pl.multiple_of(chunk_base * C, C), nch * C)],
          src_v.at[pl.ds(0, nch * C)])

      def fire_gathers(t, a):
        for k in range(K):
          off = pl.multiple_of((t * K + k) * C, C)
          pltpu.async_copy(table_hbm.at[src_v.at[pl.ds(off, C)]],
                           rows_v.at[a, pl.ds(k * C, C)], gsem[a])

      def drain_gathers(a):
        pltpu.make_async_copy(table_hbm.at[pl.ds(0, K * C)],
                              rows_v.at[a], gsem[a]).wait()

      def fire_scatters(t, a):
        for k in range(K):
          pltpu.async_copy(rows_v.at[a, pl.ds(k * C, C)],
                           acc_sh.at[dst_v.at[t * K + k]], ssem[a], add=True)

      def drain_scatters(a):
        pltpu.make_async_copy(rows_v.at[a],
                              acc_sh.at[pl.ds(0, K * C)], ssem[a]).wait()

      fire_gathers(0, 0)

      def sup_pair(i, _):
        for a in (0, 1):
          t = 2 * i + a
          b = 1 - a

          @pl.when(t + 1 < nsup)
          def _():
            @pl.when(t >= 1)
            def _():
              drain_scatters(b)
            fire_gathers(t + 1, b)

          drain_gathers(a)
          fire_scatters(t, a)
        return 0

      lax.fori_loop(0, nsup // 2, sup_pair, 0)
      drain_scatters(0)
      drain_scatters(1)

    if SLOW_CORE == 0:
      sizes = ((NCH_SLOW, 0), (NCH_FAST, NS * NCH_SLOW))
    else:
      sizes = ((NCH_FAST, 0), (NCH_SLOW, NS * NCH_FAST))
    for core_val, (nch, core_base) in enumerate(sizes):
      @pl.when(c == core_val)
      def _(nch=nch, core_base=core_base):
        run_edges(nch, core_base + s * nch)

    plsc.subcore_barrier()
    pltpu.sync_copy(acc_sh.at[pl.ds(s * RPT, RPT)],
                    out_hbm.at[c, pl.ds(s * RPT, RPT)])

  mesh = plsc.VectorSubcoreMesh(
      core_axis_name="c", subcore_axis_name="s",
      num_cores=NC, num_subcores=NS)
  kern = functools.partial(
      pl.kernel,
      out_type=jax.ShapeDtypeStruct((NC, NP, width), jnp.float32),
      mesh=mesh,
      compiler_params=pltpu.CompilerParams(use_tc_tiling_on_sc=False),
      scratch_types=[
          pltpu.VMEM((NCH_FAST * C,), jnp.int32),
          pltpu.VMEM((NCH_FAST, C), jnp.int32),
          pltpu.VMEM((2, K * C, width), jnp.float32),
          pltpu.VMEM((C, width), jnp.float32),
          pltpu.VMEM_SHARED((NP, width), jnp.float32),
          pltpu.SemaphoreType.DMA,
          pltpu.SemaphoreType.DMA,
          pltpu.SemaphoreType.DMA,
          pltpu.SemaphoreType.DMA,
      ],
  )(body)
  return kern


def _sc_count(src, dst, table):
  return _sc_edge_scatter(16, False)(src, dst, table)


def _sc_gather_scatter(src, dst, table):
  return _sc_edge_scatter(D_H, True)(src, dst, table)


_ROWS_BLK = 512
_NBLK = NP // _ROWS_BLK


def _k0_body(x_ref, w1_ref, degp_ref, h1_ref, dinv_ref):
  deg = degp_ref[0] + degp_ref[1] + 1.0
  dinv = lax.rsqrt(jnp.maximum(deg, 1.0))
  dinv_ref[...] = dinv
  h = jnp.dot(x_ref[...], w1_ref[...], preferred_element_type=jnp.float32)
  h1_ref[...] = h * dinv[:, :1]


def _kmid_body(p_ref, h_ref, dinv_ref, b_ref, w_ref, out_ref):
  dinv = dinv_ref[:, :1]
  t = dinv * (p_ref[0] + p_ref[1] + h_ref[...]) + b_ref[...]
  t = jnp.maximum(t, 0.0)
  out_ref[...] = jnp.dot(
      t, w_ref[...], preferred_element_type=jnp.float32) * dinv


def _kfinal_body(p_ref, h_ref, dinv_ref, b_ref, out_ref):
  dinv = dinv_ref[:, :1]
  t = dinv * (p_ref[0] + p_ref[1] + h_ref[...]) + b_ref[...]
  out_ref[...] = jnp.maximum(t, 0.0)


_G_PER = 8


def _kpool_body(starts_ref, t3_ref, emb_ref):
  for gg in range(_G_PER):
    g = pl.program_id(0) * _G_PER + gg
    s0 = starts_ref[g]
    e0 = starts_ref[g + 1]
    base = (s0 // 8) * 8
    nblk = (e0 - base + 7) // 8

    def body(i, carry):
      sacc, macc = carry
      r0 = base + i * 8
      rows = t3_ref[pl.ds(pl.multiple_of(r0, 8), 8), :]
      rid = r0 + lax.broadcasted_iota(jnp.int32, (8, 1), 0)
      m = (rid >= s0) & (rid < e0)
      sacc = sacc + jnp.where(m, rows, 0.0)
      macc = jnp.maximum(macc, jnp.where(m, rows, -jnp.inf))
      return sacc, macc

    init = (jnp.zeros((8, D_H), jnp.float32),
            jnp.full((8, D_H), -jnp.inf, jnp.float32))
    sacc, macc = lax.fori_loop(0, nblk, body, init)
    cnt = jnp.maximum((e0 - s0).astype(jnp.float32), 1.0)
    emb_ref[pl.ds(gg, 1), :D_H] = jnp.sum(sacc, axis=0, keepdims=True) / cnt
    emb_ref[pl.ds(gg, 1), D_H:] = jnp.max(macc, axis=0, keepdims=True)


def _kmlp_body(emb_ref, wc1_ref, bc1_ref, s1_ref, t1_ref,
               wc2_ref, bc2_ref, s2_ref, t2_ref, wc3_ref, bc3_ref, out_ref):
  z = jnp.dot(emb_ref[...], wc1_ref[...], preferred_element_type=jnp.float32)
  z = jnp.maximum(z * s1_ref[...] + t1_ref[...], 0.0)
  z = jnp.dot(z, wc2_ref[...], preferred_element_type=jnp.float32)
  z = jnp.maximum(z * s2_ref[...] + t2_ref[...], 0.0)
  out_ref[...] = jnp.dot(
      z, wc3_ref[...], preferred_element_type=jnp.float32) + bc3_ref[...]


def kernel(x, edge_index, batch, W1, b1, W2, b2, W3, b3, Wc1, bc1, g1, be1,
           m1, v1, Wc2, bc2, g2, be2, m2, v2, Wc3, bc3):
  f32 = jnp.float32
  pad_e = EP - E
  pad_dst = N + jnp.arange(pad_e, dtype=jnp.int32) % (NP - N)
  src = jnp.concatenate([edge_index[0], jnp.zeros((pad_e,), jnp.int32)])
  dst = jnp.concatenate(
      [edge_index[1], pad_dst]).reshape(TOTCH, C)
  xp = jnp.pad(x, ((0, NP - N), (0, 0)))
  starts = jnp.searchsorted(batch, jnp.arange(G + 1, dtype=jnp.int32)
                            ).astype(jnp.int32)

  degp = _sc_count(src, dst, jnp.zeros((NP, 16), f32))

  blk = _ROWS_BLK
  h1, dinv = pl.pallas_call(
      _k0_body,
      grid=(_NBLK,),
      in_specs=[
          pl.BlockSpec((blk, D_IN), lambda i: (i, 0)),
          pl.BlockSpec((D_IN, D_H), lambda i: (0, 0)),
          pl.BlockSpec((NC, blk, 16), lambda i: (0, i, 0)),
      ],
      out_specs=[
          pl.BlockSpec((blk, D_H), lambda i: (i, 0)),
          pl.BlockSpec((blk, 16), lambda i: (i, 0)),
      ],
      out_shape=[
          jax.ShapeDtypeStruct((NP, D_H), f32),
          jax.ShapeDtypeStruct((NP, 16), f32),
      ],
  )(xp, W1, degp)

  def mid(h, b, w):
    p = _sc_gather_scatter(src, dst, h)
    return pl.pallas_call(
        _kmid_body,
        grid=(_NBLK,),
        in_specs=[
            pl.BlockSpec((NC, blk, D_H), lambda i: (0, i, 0)),
            pl.BlockSpec((blk, D_H), lambda i: (i, 0)),
            pl.BlockSpec((blk, 16), lambda i: (i, 0)),
            pl.BlockSpec((1, D_H), lambda i: (0, 0)),
            pl.BlockSpec((D_H, D_H), lambda i: (0, 0)),
        ],
        out_specs=pl.BlockSpec((blk, D_H), lambda i: (i, 0)),
        out_shape=jax.ShapeDtypeStruct((NP, D_H), f32),
    )(p, h, dinv, b.reshape(1, D_H), w)

  h2 = mid(h1, b1, W2)
  h3 = mid(h2, b2, W3)

  p3 = _sc_gather_scatter(src, dst, h3)
  t3 = pl.pallas_call(
      _kfinal_body,
      grid=(_NBLK,),
      in_specs=[
          pl.BlockSpec((NC, blk, D_H), lambda i: (0, i, 0)),
          pl.BlockSpec((blk, D_H), lambda i: (i, 0)),
          pl.BlockSpec((blk, 16), lambda i: (i, 0)),
          pl.BlockSpec((1, D_H), lambda i: (0, 0)),
      ],
      out_specs=pl.BlockSpec((blk, D_H), lambda i: (i, 0)),
      out_shape=jax.ShapeDtypeStruct((NP, D_H), f32),
  )(p3, h3, dinv, b3.reshape(1, D_H))

  emb = pl.pallas_call(
      _kpool_body,
      grid=(G // _G_PER,),
      in_specs=[
          pl.BlockSpec(memory_space=pltpu.SMEM),
          pl.BlockSpec((NP, D_H), lambda g: (0, 0)),
      ],
      out_specs=pl.BlockSpec((_G_PER, 2 * D_H), lambda g: (g, 0)),
      out_shape=jax.ShapeDtypeStruct((G, 2 * D_H), f32),
  )(starts, t3)

  s1 = (g1 * lax.rsqrt(v1 + EPS)).reshape(1, D_H)
  t1 = (be1 - m1 * g1 * lax.rsqrt(v1 + EPS) + bc1 * g1 * lax.rsqrt(v1 + EPS)
        ).reshape(1, D_H)
  s2 = (g2 * lax.rsqrt(v2 + EPS)).reshape(1, D_H // 2)
  t2 = (be2 - m2 * g2 * lax.rsqrt(v2 + EPS) + bc2 * g2 * lax.rsqrt(v2 + EPS)
        ).reshape(1, D_H // 2)

  full = lambda shape: pl.BlockSpec(shape, lambda: (0,) * len(shape))
  out = pl.pallas_call(
      _kmlp_body,
      in_specs=[
          full((G, 2 * D_H)), full((2 * D_H, D_H)), full((1, D_H)),
          full((1, D_H)), full((1, D_H)),
          full((D_H, D_H // 2)), full((1, D_H // 2)), full((1, D_H // 2)),
          full((1, D_H // 2)), full((D_H // 2, T)), full((1, T)),
      ],
      out_specs=full((G, T)),
      out_shape=jax.ShapeDtypeStruct((G, T), f32),
  )(emb, Wc1, bc1.reshape(1, D_H), s1, t1,
    Wc2, bc2.reshape(1, D_H // 2), s2, t2, Wc3, bc3.reshape(1, T))
  return out
